# Initial kernel scaffold; baseline (speedup 1.0000x reference)
import jax
import jax.numpy as jnp
from jax.experimental import pallas as pl


def kernel(query_len, key_len, bias_embedding_table):
    raise NotImplementedError("write your pallas kernel here")


# speedup vs baseline: 1354.5221x; 1354.5221x over previous
import functools

import jax
import jax.numpy as jnp
from jax import lax
from jax.experimental import pallas as pl
from jax.experimental.pallas import tpu as pltpu
from jax.experimental.pallas import tpu_sc as plsc

_NSHIFT = 8
_W = 8208


def _build_sc_call(n, num_cores, num_subcores):
    nw = num_cores * num_subcores
    rpw = n // nw
    groups = rpw // _NSHIFT
    mesh = plsc.VectorSubcoreMesh(core_axis_name="c", subcore_axis_name="s")

    @functools.partial(
        pl.kernel,
        mesh=mesh,
        compiler_params=pltpu.CompilerParams(use_tc_tiling_on_sc=False),
        out_type=jax.ShapeDtypeStruct((n, n), jnp.float32),
        scratch_types=[
            pltpu.VMEM((_NSHIFT, _W), jnp.float32),
            pltpu.SemaphoreType.DMA,
            pltpu.SemaphoreType.DMA,
        ],
    )
    def run(fs_hbm, out_hbm, fs_v, load_sem, row_sem):
        wid = lax.axis_index("s") * num_cores + lax.axis_index("c")
        base = wid * rpw
        pltpu.async_copy(fs_hbm, fs_v, load_sem).wait()
        descs = []
        for g in range(groups):
            row0 = base + g * _NSHIFT
            q = (n - _NSHIFT) - row0
            descs.append(
                pltpu.async_copy(
                    fs_v.at[:, pl.ds(q, n)],
                    out_hbm.at[pl.ds(row0, _NSHIFT)],
                    row_sem,
                )
            )
        for d in descs:
            d.wait()

    return run


def kernel(query_len, key_len, bias_embedding_table):
    n = bias_embedding_table.shape[0]
    flat = bias_embedding_table[:, 0]
    f_ext = jnp.concatenate(
        [flat[::-1], jnp.full((_W + _NSHIFT - n,), flat[0], flat.dtype)]
    )
    fs = jnp.stack([f_ext[_NSHIFT - 1 - b:][:_W] for b in range(_NSHIFT)])
    info = plsc.get_sparse_core_info()
    run = _build_sc_call(n, info.num_cores, info.num_subcores)
    return run(fs.astype(jnp.float32))
